# baseline (speedup 1.0000x reference)
import functools

import jax
import jax.numpy as jnp
from jax.experimental import pallas as pl


def _gating_kernel(x_ref, gw_ref, gb_ref, comb_ref):
    logits = jnp.dot(x_ref[...], gw_ref[...].T,
                     preferred_element_type=jnp.float32) + gb_ref[0][None, :]
    E = logits.shape[1]
    col = jax.lax.broadcasted_iota(jnp.int32, logits.shape, 1)
    a1 = jnp.argmax(logits, axis=1, keepdims=True)
    m1 = jnp.max(logits, axis=1, keepdims=True)
    masked = jnp.where(col == a1, -jnp.inf, logits)
    a2 = jnp.argmax(masked, axis=1, keepdims=True)
    m2 = jnp.max(masked, axis=1, keepdims=True)
    z = jnp.exp(m2 - m1)
    w1 = 1.0 / (1.0 + z)
    w2 = z / (1.0 + z)
    comb_ref[...] = jnp.where(col == a1, w1, 0.0) + jnp.where(col == a2, w2, 0.0)


def _expert_kernel(comb_ref, x_ref, w_ref, b_ref, out_ref):
    e = pl.program_id(2)

    @pl.when(e == 0)
    def _init():
        out_ref[...] = jnp.zeros_like(out_ref)

    comb = comb_ref[...]
    col = jax.lax.broadcasted_iota(jnp.int32, comb.shape, 1)
    w_e = jnp.sum(jnp.where(col == e, comb, 0.0), axis=1, keepdims=True)
    part = jnp.dot(x_ref[...], w_ref[0].T,
                   preferred_element_type=jnp.float32) + b_ref[0]
    out_ref[...] += w_e * part


@jax.jit
def kernel(x, gate_W, gate_b, W, b):
    B, D = x.shape
    E, H, _ = W.shape
    bB = min(B, 2048)
    bH = min(H, 512)
    gB = min(B, 512)

    comb = pl.pallas_call(
        _gating_kernel,
        grid=(B // gB,),
        in_specs=[
            pl.BlockSpec((gB, D), lambda i: (i, 0)),
            pl.BlockSpec((E, D), lambda i: (0, 0)),
            pl.BlockSpec((1, E), lambda i: (0, 0)),
        ],
        out_specs=pl.BlockSpec((gB, E), lambda i: (i, 0)),
        out_shape=jax.ShapeDtypeStruct((B, E), jnp.float32),
    )(x, gate_W, gate_b.reshape(1, E))

    out = pl.pallas_call(
        _expert_kernel,
        grid=(B // bB, H // bH, E),
        in_specs=[
            pl.BlockSpec((bB, E), lambda i, j, e: (i, 0)),
            pl.BlockSpec((bB, D), lambda i, j, e: (i, 0)),
            pl.BlockSpec((1, bH, D), lambda i, j, e: (e, j, 0)),
            pl.BlockSpec((1, 1, bH), lambda i, j, e: (e, 0, j)),
        ],
        out_specs=pl.BlockSpec((bB, bH), lambda i, j, e: (i, j)),
        out_shape=jax.ShapeDtypeStruct((B, H), jnp.float32),
    )(comb, x, W, b.reshape(E, 1, H))
    return out
